# Initial kernel scaffold; baseline (speedup 1.0000x reference)
import jax
import jax.numpy as jnp
from jax.experimental import pallas as pl


def kernel(x, e_i_ts):
    raise NotImplementedError("write your pallas kernel here")


# speedup vs baseline: 1.8097x; 1.8097x over previous
import functools

import jax
import jax.numpy as jnp
from jax.experimental import pallas as pl
from jax.experimental.pallas import tpu as pltpu


def _vq_block(x_ref, e_ref, out_ref, ind_ref):
    x_blk = x_ref[0]
    e = e_ref[...]

    scores = jax.lax.dot_general(
        x_blk, e, (((0,), (0,)), ((), ())),
        preferred_element_type=jnp.float32)
    xn2 = jnp.sum(x_blk * x_blk, axis=0)
    en2 = jnp.sum(e * e, axis=0)
    dist = (xn2[:, None] - 2.0 * scores) + en2[None, :]
    idx = jnp.argmin(dist, axis=1).astype(jnp.int32)

    k_iota = jax.lax.broadcasted_iota(jnp.int32, dist.shape, 1)
    onehot = (k_iota == idx[:, None]).astype(jnp.float32)
    q = jax.lax.dot_general(
        e, onehot, (((1,), (1,)), ((), ())),
        preferred_element_type=jnp.float32)

    def _norm0(v):
        return jnp.maximum(
            jnp.sqrt(jnp.sum(v * v, axis=0, keepdims=True)), 1e-06)

    e_norm = _norm0(x_blk)
    q_norm = _norm0(q)
    e_hat = x_blk / e_norm
    q_hat = q / q_norm
    lambda_val = q_norm / e_norm
    s = e_hat + q_hat
    r = s / _norm0(s)
    r_dot_e = jnp.sum(r * x_blk, axis=0, keepdims=True)
    out = lambda_val * (x_blk - 2.0 * (r * r_dot_e) + 2.0 * (q_hat * e_norm))

    out_ref[0] = out
    ind_ref[0, 0] = idx


@functools.partial(jax.jit, static_argnames=())
def kernel(x, e_i_ts):
    B, C, H, W = x.shape
    K = e_i_ts.shape[1]
    T = H * W
    x3 = x.reshape(B, C, T)

    out, ind = pl.pallas_call(
        _vq_block,
        grid=(B,),
        in_specs=[
            pl.BlockSpec((1, C, T), lambda b: (b, 0, 0)),
            pl.BlockSpec((C, K), lambda b: (0, 0)),
        ],
        out_specs=[
            pl.BlockSpec((1, C, T), lambda b: (b, 0, 0)),
            pl.BlockSpec((1, 1, T), lambda b: (b, 0, 0)),
        ],
        out_shape=[
            jax.ShapeDtypeStruct((B, C, T), jnp.float32),
            jax.ShapeDtypeStruct((B, 1, T), jnp.int32),
        ],
    )(x3, e_i_ts)

    return out.reshape(B, C, H, W), ind.reshape(B, H, W)


# speedup vs baseline: 1.9398x; 1.0719x over previous
import jax
import jax.numpy as jnp
from jax.experimental import pallas as pl
from jax.experimental.pallas import tpu as pltpu


def _vq_block(x_ref, e_ref, out_ref, ind_ref, es_ref, en2_ref):
    @pl.when(pl.program_id(0) == 0)
    def _():
        e = e_ref[...]
        es_ref[...] = e * (-2.0)
        en2_ref[...] = jnp.sum(e * e, axis=0, keepdims=True)

    x_blk = x_ref[0]
    es = es_ref[...]

    scores2 = jax.lax.dot_general(
        x_blk, es, (((0,), (0,)), ((), ())),
        preferred_element_type=jnp.float32)
    dist = scores2 + en2_ref[0][None, :]
    idx = jnp.argmin(dist, axis=1).astype(jnp.int32)

    k_iota = jax.lax.broadcasted_iota(jnp.int32, dist.shape, 1)
    onehot = jnp.where(k_iota == idx[:, None], -0.5, 0.0)
    q = jax.lax.dot_general(
        es, onehot, (((1,), (1,)), ((), ())),
        preferred_element_type=jnp.float32)

    def _norm0(v):
        return jnp.maximum(
            jnp.sqrt(jnp.sum(v * v, axis=0, keepdims=True)), 1e-06)

    e_norm = _norm0(x_blk)
    q_norm = _norm0(q)
    e_hat = x_blk / e_norm
    q_hat = q / q_norm
    lambda_val = q_norm / e_norm
    s = e_hat + q_hat
    r = s / _norm0(s)
    r_dot_e = jnp.sum(r * x_blk, axis=0, keepdims=True)
    out = lambda_val * (x_blk - 2.0 * (r * r_dot_e) + 2.0 * (q_hat * e_norm))

    out_ref[0] = out
    ind_ref[0, 0] = idx


@jax.jit
def kernel(x, e_i_ts):
    B, C, H, W = x.shape
    K = e_i_ts.shape[1]
    T = H * W
    x3 = x.reshape(B, C, T)

    out, ind = pl.pallas_call(
        _vq_block,
        grid=(B,),
        in_specs=[
            pl.BlockSpec((1, C, T), lambda b: (b, 0, 0)),
            pl.BlockSpec((C, K), lambda b: (0, 0)),
        ],
        out_specs=[
            pl.BlockSpec((1, C, T), lambda b: (b, 0, 0)),
            pl.BlockSpec((1, 1, T), lambda b: (b, 0, 0)),
        ],
        out_shape=[
            jax.ShapeDtypeStruct((B, C, T), jnp.float32),
            jax.ShapeDtypeStruct((B, 1, T), jnp.int32),
        ],
        scratch_shapes=[
            pltpu.VMEM((C, K), jnp.float32),
            pltpu.VMEM((1, K), jnp.float32),
        ],
    )(x3, e_i_ts)

    return out.reshape(B, C, H, W), ind.reshape(B, H, W)


# speedup vs baseline: 2.9901x; 1.5415x over previous
import jax
import jax.numpy as jnp
from jax.experimental import pallas as pl
from jax.experimental.pallas import tpu as pltpu

TB = 1024


def _vq_block(x_ref, e_ref, out_ref, ind_ref, es_ref, en2_ref):
    @pl.when(pl.program_id(0) == 0)
    def _():
        e = e_ref[...]
        es_ref[...] = e * (-2.0)
        en2_ref[...] = jnp.sum(e * e, axis=0, keepdims=True)

    x_blk = x_ref[...]
    es = es_ref[...]

    scores2 = jnp.dot(x_blk, es, preferred_element_type=jnp.float32)
    dist = scores2 + en2_ref[...]
    idx = jnp.argmin(dist, axis=1).astype(jnp.int32)

    k_iota = jax.lax.broadcasted_iota(jnp.int32, dist.shape, 1)
    onehot = jnp.where(k_iota == idx[:, None], -0.5, 0.0)
    q = jax.lax.dot_general(
        onehot, es, (((1,), (1,)), ((), ())),
        preferred_element_type=jnp.float32)

    def _norm1(v):
        return jnp.maximum(
            jnp.sqrt(jnp.sum(v * v, axis=1, keepdims=True)), 1e-06)

    e_norm = _norm1(x_blk)
    q_norm = _norm1(q)
    e_hat = x_blk / e_norm
    q_hat = q / q_norm
    lambda_val = q_norm / e_norm
    s = e_hat + q_hat
    r = s / _norm1(s)
    r_dot_e = jnp.sum(r * x_blk, axis=1, keepdims=True)
    out = lambda_val * (x_blk - 2.0 * (r * r_dot_e) + 2.0 * (q_hat * e_norm))

    out_ref[...] = out
    ind_ref[0, 0] = idx


@jax.jit
def kernel(x, e_i_ts):
    B, C, H, W = x.shape
    K = e_i_ts.shape[1]
    Ttot = B * H * W
    NB = Ttot // TB
    x_tok = jnp.transpose(x, (0, 2, 3, 1)).reshape(Ttot, C)

    out, ind = pl.pallas_call(
        _vq_block,
        grid=(NB,),
        in_specs=[
            pl.BlockSpec((TB, C), lambda t: (t, 0)),
            pl.BlockSpec((C, K), lambda t: (0, 0)),
        ],
        out_specs=[
            pl.BlockSpec((TB, C), lambda t: (t, 0)),
            pl.BlockSpec((1, 1, TB), lambda t: (t, 0, 0)),
        ],
        out_shape=[
            jax.ShapeDtypeStruct((Ttot, C), jnp.float32),
            jax.ShapeDtypeStruct((NB, 1, TB), jnp.int32),
        ],
        scratch_shapes=[
            pltpu.VMEM((C, K), jnp.float32),
            pltpu.VMEM((1, K), jnp.float32),
        ],
    )(x_tok, e_i_ts)

    quant = jnp.transpose(out.reshape(B, H, W, C), (0, 3, 1, 2))
    return quant, ind.reshape(B, H, W)


# speedup vs baseline: 3.1606x; 1.0570x over previous
import jax
import jax.numpy as jnp
from jax.experimental import pallas as pl
from jax.experimental.pallas import tpu as pltpu

TB = 2048


def _vq_block(x_ref, e_ref, out_ref, ind_ref, es_ref, en2_ref):
    @pl.when(pl.program_id(0) == 0)
    def _():
        e = e_ref[...]
        es_ref[...] = e * (-2.0)
        en2_ref[...] = jnp.sum(e * e, axis=0, keepdims=True)

    x_blk = x_ref[...]
    es = es_ref[...]

    scores2 = jnp.dot(x_blk, es, preferred_element_type=jnp.float32)
    dist = scores2 + en2_ref[...]
    idx = jnp.argmin(dist, axis=1).astype(jnp.int32)

    k_iota = jax.lax.broadcasted_iota(jnp.int32, dist.shape, 1)
    onehot = jnp.where(k_iota == idx[:, None], -0.5, 0.0)
    q = jax.lax.dot_general(
        onehot, es, (((1,), (1,)), ((), ())),
        preferred_element_type=jnp.float32)

    def _norm1(v):
        return jnp.maximum(
            jnp.sqrt(jnp.sum(v * v, axis=1, keepdims=True)), 1e-06)

    e_norm = _norm1(x_blk)
    q_norm = _norm1(q)
    e_hat = x_blk / e_norm
    q_hat = q / q_norm
    lambda_val = q_norm / e_norm
    s = e_hat + q_hat
    r = s / _norm1(s)
    r_dot_e = jnp.sum(r * x_blk, axis=1, keepdims=True)
    out = lambda_val * (x_blk - 2.0 * (r * r_dot_e) + 2.0 * (q_hat * e_norm))

    out_ref[...] = out
    ind_ref[0, 0] = idx


@jax.jit
def kernel(x, e_i_ts):
    B, C, H, W = x.shape
    K = e_i_ts.shape[1]
    Ttot = B * H * W
    NB = Ttot // TB
    x_tok = jnp.transpose(x, (0, 2, 3, 1)).reshape(Ttot, C)

    out, ind = pl.pallas_call(
        _vq_block,
        grid=(NB,),
        in_specs=[
            pl.BlockSpec((TB, C), lambda t: (t, 0)),
            pl.BlockSpec((C, K), lambda t: (0, 0)),
        ],
        out_specs=[
            pl.BlockSpec((TB, C), lambda t: (t, 0)),
            pl.BlockSpec((1, 1, TB), lambda t: (t, 0, 0)),
        ],
        out_shape=[
            jax.ShapeDtypeStruct((Ttot, C), jnp.float32),
            jax.ShapeDtypeStruct((NB, 1, TB), jnp.int32),
        ],
        scratch_shapes=[
            pltpu.VMEM((C, K), jnp.float32),
            pltpu.VMEM((1, K), jnp.float32),
        ],
    )(x_tok, e_i_ts)

    quant = jnp.transpose(out.reshape(B, H, W, C), (0, 3, 1, 2))
    return quant, ind.reshape(B, H, W)


# speedup vs baseline: 3.4927x; 1.1051x over previous
import jax
import jax.numpy as jnp
from jax.experimental import pallas as pl
from jax.experimental.pallas import tpu as pltpu

TB = 2048


def _vq_block(x_ref, e_ref, out_ref, ind_ref, es_ref, en2_ref):
    @pl.when(pl.program_id(0) == 0)
    def _():
        e = e_ref[...]
        es_ref[...] = e * (-2.0)
        en2_ref[...] = jnp.sum(e * e, axis=0, keepdims=True)

    x_blk = x_ref[...]
    es = es_ref[...]

    scores2 = jnp.dot(x_blk, es, preferred_element_type=jnp.float32)
    dist = scores2 + en2_ref[...]
    idx = jnp.argmin(dist, axis=1).astype(jnp.int32)

    k_iota = jax.lax.broadcasted_iota(jnp.int32, dist.shape, 1)
    onehot = jnp.where(k_iota == idx[:, None], -0.5, 0.0)
    q = jax.lax.dot_general(
        onehot, es, (((1,), (1,)), ((), ())),
        preferred_element_type=jnp.float32)

    xx = jnp.sum(x_blk * x_blk, axis=1).reshape(TB // 128, 128)
    qq = jnp.sum(q * q, axis=1).reshape(TB // 128, 128)
    xq = jnp.sum(x_blk * q, axis=1).reshape(TB // 128, 128)

    e_inv = jnp.minimum(jax.lax.rsqrt(xx), 1e+06)
    q_inv = jnp.minimum(jax.lax.rsqrt(qq), 1e+06)
    e_norm = xx * e_inv
    q_norm = qq * q_inv
    lam = q_norm * e_inv
    c = xq * (e_inv * q_inv)
    ss = xx * (e_inv * e_inv) + 2.0 * c + qq * (q_inv * q_inv)
    ns_inv = jnp.minimum(jax.lax.rsqrt(ss), 1e+06)
    r_dot_e = (e_norm + xq * q_inv) * ns_inv
    a = lam * (1.0 - 2.0 * r_dot_e * ns_inv * e_inv)
    b = (lam * q_inv) * (2.0 * e_norm - 2.0 * r_dot_e * ns_inv)

    a_col = a.reshape(TB, 1)
    b_col = b.reshape(TB, 1)
    out_ref[...] = a_col * x_blk + b_col * q
    ind_ref[0, 0] = idx


@jax.jit
def kernel(x, e_i_ts):
    B, C, H, W = x.shape
    K = e_i_ts.shape[1]
    Ttot = B * H * W
    NB = Ttot // TB
    x_tok = jnp.transpose(x, (0, 2, 3, 1)).reshape(Ttot, C)

    out, ind = pl.pallas_call(
        _vq_block,
        grid=(NB,),
        in_specs=[
            pl.BlockSpec((TB, C), lambda t: (t, 0)),
            pl.BlockSpec((C, K), lambda t: (0, 0)),
        ],
        out_specs=[
            pl.BlockSpec((TB, C), lambda t: (t, 0)),
            pl.BlockSpec((1, 1, TB), lambda t: (t, 0, 0)),
        ],
        out_shape=[
            jax.ShapeDtypeStruct((Ttot, C), jnp.float32),
            jax.ShapeDtypeStruct((NB, 1, TB), jnp.int32),
        ],
        scratch_shapes=[
            pltpu.VMEM((C, K), jnp.float32),
            pltpu.VMEM((1, K), jnp.float32),
        ],
    )(x_tok, e_i_ts)

    quant = jnp.transpose(out.reshape(B, H, W, C), (0, 3, 1, 2))
    return quant, ind.reshape(B, H, W)
